# baseline (speedup 1.0000x reference)
import functools

import jax
import jax.numpy as jnp
from jax import lax
from jax.experimental import pallas as pl

_K = 32
_C = 128


def _key(v):
    u = lax.bitcast_convert_type(v, jnp.int32)
    return u ^ (jnp.right_shift(u, 31) & jnp.int32(0x7FFFFFFF))


def _unkey(k):
    return lax.bitcast_convert_type(
        k ^ (jnp.right_shift(k, 31) & jnp.int32(0x7FFFFFFF)), jnp.float32)


def _enc_body(x_ref, bpre_ref, w_ref, benc_ref, out_ref):
    xc = x_ref[...] - bpre_ref[...]
    acc = lax.dot_general(
        xc, w_ref[...],
        dimension_numbers=(((1,), (1,)), ((), ())),
        precision=lax.Precision.DEFAULT,
        preferred_element_type=jnp.float32,
    ) + benc_ref[...]
    tb = acc.shape[0]
    out_ref[...] = acc.reshape(tb, 8, 128)


def _thr_body(pre_ref, thr_ref, *, k):
    p = pre_ref[...]
    rows = p.shape[0]
    cmax = jnp.max(p, axis=-1)
    ckey = _key(cmax)
    rmax = jnp.max(ckey, axis=-1, keepdims=True)

    lo = jnp.full((rows, 1), jnp.iinfo(jnp.int32).min, jnp.int32)
    hi = rmax

    def cstep(_, carry):
        lo, hi = carry
        xo = lo ^ hi
        mid = (lo & hi) + jnp.right_shift(xo, 1) + (xo & 1)
        cnt = jnp.sum((ckey >= mid).astype(jnp.int32), axis=1, keepdims=True)
        p_ = cnt >= k
        return jnp.where(p_, mid, lo), jnp.where(p_, hi, mid - 1)

    lo, hi = lax.fori_loop(0, 33, cstep, (lo, hi))

    key = _key(p)
    lo = lo
    hi = rmax

    def cond(carry):
        i, lo, hi = carry
        return (i < jnp.int32(34)) & jnp.any(hi > lo)

    def step(carry):
        i, lo, hi = carry
        xo = lo ^ hi
        mid = (lo & hi) + jnp.right_shift(xo, 1) + (xo & 1)
        cnt = jnp.sum((key >= mid.reshape(rows, 1, 1)).astype(jnp.int32),
                      axis=(1, 2)).reshape(rows, 1)
        p_ = cnt >= k
        return i + 1, jnp.where(p_, mid, lo), jnp.where(p_, hi, mid - 1)

    _, lo, hi = lax.while_loop(cond, step, (jnp.int32(0), lo, hi))
    thr_ref[...] = jnp.broadcast_to(_unkey(lo), thr_ref.shape)


def _dec_body(pre_ref, thr_ref, w_ref, bdec_ref, bpre_ref,
              hid_ref, out_ref, l0_ref):
    t = pl.program_id(0)
    kk = pl.program_id(1)
    nk = pl.num_programs(1)
    p3 = pre_ref[...]
    td = p3.shape[0]
    pm = p3.reshape(td, 1024)
    thr = thr_ref[:, 0:1]
    mask = pm >= thr
    hid = jnp.where(mask, jnp.maximum(pm, 0.0), 0.0)
    hid_ref[...] = hid

    part = lax.dot_general(
        hid.astype(jnp.bfloat16), w_ref[...],
        dimension_numbers=(((1,), (1,)), ((), ())),
        precision=lax.Precision.DEFAULT,
        preferred_element_type=jnp.float32,
    )

    @pl.when(kk == 0)
    def _():
        out_ref[...] = part

    @pl.when(kk > 0)
    def _():
        out_ref[...] += part

    @pl.when((t == 0) & (kk == 0))
    def _():
        l0_ref[...] = jnp.zeros_like(l0_ref)

    l0_ref[...] += jnp.full(
        (1, 1), jnp.sum((mask & (pm > 0.0)).astype(jnp.float32)), jnp.float32)

    @pl.when(kk == nk - 1)
    def _():
        out_ref[...] += bdec_ref[...] + bpre_ref[...]


def _loss_body(rec_ref, x_ref, loss_ref):
    t = pl.program_id(0)

    @pl.when(t == 0)
    def _():
        loss_ref[...] = jnp.zeros_like(loss_ref)

    dd = rec_ref[...] - x_ref[...]
    loss_ref[...] += jnp.full((1, 1), jnp.sum(dd * dd), jnp.float32)


def kernel(x, b_pre, W_enc, b_enc, W_dec, b_dec):
    n, d = x.shape
    h = W_enc.shape[0]
    f32 = jnp.float32
    nch = h // _C

    bpre2 = b_pre.reshape(1, d)
    benc2 = b_enc.reshape(1, h)
    bdec2 = b_dec.reshape(1, d)
    wdec_bf16 = W_dec.astype(jnp.bfloat16)

    tb, hb = 1024, 1024
    pre_v = pl.pallas_call(
        _enc_body,
        grid=(n // tb, h // hb),
        in_specs=[
            pl.BlockSpec((tb, d), lambda t, hh: (t, 0)),
            pl.BlockSpec((1, d), lambda t, hh: (0, 0)),
            pl.BlockSpec((hb, d), lambda t, hh: (hh, 0)),
            pl.BlockSpec((1, hb), lambda t, hh: (0, hh)),
        ],
        out_specs=pl.BlockSpec((tb, 8, 128), lambda t, hh: (t, hh, 0)),
        out_shape=jax.ShapeDtypeStruct((n, nch, _C), f32),
    )(x, bpre2, W_enc, benc2)

    rb = 128
    thr = pl.pallas_call(
        functools.partial(_thr_body, k=_K),
        grid=(n // rb,),
        in_specs=[pl.BlockSpec((rb, nch, _C), lambda t: (t, 0, 0))],
        out_specs=pl.BlockSpec((rb, _C), lambda t: (t, 0)),
        out_shape=jax.ShapeDtypeStruct((n, _C), f32),
    )(pre_v)

    td, kb = 512, 1024
    hidden, recon, l0_sum = pl.pallas_call(
        _dec_body,
        grid=(n // td, h // kb),
        in_specs=[
            pl.BlockSpec((td, 8, 128), lambda t, kk: (t, kk, 0)),
            pl.BlockSpec((td, _C), lambda t, kk: (t, 0)),
            pl.BlockSpec((d, kb), lambda t, kk: (0, kk)),
            pl.BlockSpec((1, d), lambda t, kk: (0, 0)),
            pl.BlockSpec((1, d), lambda t, kk: (0, 0)),
        ],
        out_specs=[
            pl.BlockSpec((td, kb), lambda t, kk: (t, kk)),
            pl.BlockSpec((td, d), lambda t, kk: (t, 0)),
            pl.BlockSpec((1, 1), lambda t, kk: (0, 0)),
        ],
        out_shape=[
            jax.ShapeDtypeStruct((n, h), f32),
            jax.ShapeDtypeStruct((n, d), f32),
            jax.ShapeDtypeStruct((1, 1), f32),
        ],
    )(pre_v, thr, wdec_bf16, bdec2, bpre2)

    tl = 1024
    loss_sum = pl.pallas_call(
        _loss_body,
        grid=(n // tl,),
        in_specs=[
            pl.BlockSpec((tl, d), lambda t: (t, 0)),
            pl.BlockSpec((tl, d), lambda t: (t, 0)),
        ],
        out_specs=pl.BlockSpec((1, 1), lambda t: (0, 0)),
        out_shape=jax.ShapeDtypeStruct((1, 1), f32),
    )(recon, x)

    rec_loss = loss_sum[0, 0] / jnp.float32(n * d)
    l0 = l0_sum[0, 0] / jnp.float32(n)
    sparsity = jnp.zeros((), f32)
    return (recon, hidden, rec_loss, rec_loss, sparsity, l0)


# speedup vs baseline: 2.4443x; 2.4443x over previous
import functools

import jax
import jax.numpy as jnp
from jax import lax
from jax.experimental import pallas as pl

_K = 32


def _enc_body(x_ref, bpre_ref, w_ref, benc_ref, out_ref):
    xc = x_ref[...] - bpre_ref[...]
    out_ref[...] = lax.dot_general(
        xc, w_ref[...],
        dimension_numbers=(((1,), (1,)), ((), ())),
        precision=lax.Precision.DEFAULT,
        preferred_element_type=jnp.float32,
    ) + benc_ref[...]


def _topk_body(pre_ref, hid_ref, l0_ref, *, k):
    t = pl.program_id(0)
    v = pre_ref[...]
    u = lax.bitcast_convert_type(v, jnp.int32)
    key = u ^ (jnp.right_shift(u, 31) & jnp.int32(0x7FFFFFFF))

    rows = v.shape[0]
    lo = jnp.full((rows, 1), jnp.iinfo(jnp.int32).min, jnp.int32)
    hi = jnp.full((rows, 1), jnp.iinfo(jnp.int32).max, jnp.int32)

    def step(_, carry):
        lo, hi = carry
        xo = lo ^ hi
        mid = (lo & hi) + jnp.right_shift(xo, 1) + (xo & 1)
        cnt = jnp.sum((key >= mid).astype(jnp.int32), axis=1, keepdims=True)
        p = cnt >= k
        return jnp.where(p, mid, lo), jnp.where(p, hi, mid - 1)

    lo, hi = lax.fori_loop(0, 33, step, (lo, hi))
    mask = key >= lo
    hid_ref[...] = jnp.where(mask, jnp.maximum(v, 0.0), 0.0)

    @pl.when(t == 0)
    def _():
        l0_ref[...] = jnp.zeros_like(l0_ref)

    pos = jnp.sum((mask & (v > 0.0)).astype(jnp.float32))
    l0_ref[...] += jnp.full((1, 1), pos, jnp.float32)


def _dec_body(hid_ref, w_ref, bdec_ref, bpre_ref, out_ref):
    kk = pl.program_id(1)
    nk = pl.num_programs(1)
    part = lax.dot_general(
        hid_ref[...].astype(jnp.bfloat16), w_ref[...],
        dimension_numbers=(((1,), (1,)), ((), ())),
        precision=lax.Precision.DEFAULT,
        preferred_element_type=jnp.float32,
    )

    @pl.when(kk == 0)
    def _():
        out_ref[...] = part

    @pl.when(kk > 0)
    def _():
        out_ref[...] += part

    @pl.when(kk == nk - 1)
    def _():
        out_ref[...] += bdec_ref[...] + bpre_ref[...]


def _loss_body(rec_ref, x_ref, loss_ref):
    t = pl.program_id(0)

    @pl.when(t == 0)
    def _():
        loss_ref[...] = jnp.zeros_like(loss_ref)

    dd = rec_ref[...] - x_ref[...]
    loss_ref[...] += jnp.full((1, 1), jnp.sum(dd * dd), jnp.float32)


def kernel(x, b_pre, W_enc, b_enc, W_dec, b_dec):
    n, d = x.shape
    h = W_enc.shape[0]
    f32 = jnp.float32

    bpre2 = b_pre.reshape(1, d)
    benc2 = b_enc.reshape(1, h)
    bdec2 = b_dec.reshape(1, d)
    wdec_bf16 = W_dec.astype(jnp.bfloat16)

    tb = min(1024, n)
    hb = min(1024, h)
    pre = pl.pallas_call(
        _enc_body,
        grid=(n // tb, h // hb),
        in_specs=[
            pl.BlockSpec((tb, d), lambda t, hh: (t, 0)),
            pl.BlockSpec((1, d), lambda t, hh: (0, 0)),
            pl.BlockSpec((hb, d), lambda t, hh: (hh, 0)),
            pl.BlockSpec((1, hb), lambda t, hh: (0, hh)),
        ],
        out_specs=pl.BlockSpec((tb, hb), lambda t, hh: (t, hh)),
        out_shape=jax.ShapeDtypeStruct((n, h), f32),
    )(x, bpre2, W_enc, benc2)

    rb = min(128, n)
    hidden, l0_sum = pl.pallas_call(
        functools.partial(_topk_body, k=_K),
        grid=(n // rb,),
        in_specs=[pl.BlockSpec((rb, h), lambda t: (t, 0))],
        out_specs=[
            pl.BlockSpec((rb, h), lambda t: (t, 0)),
            pl.BlockSpec((1, 1), lambda t: (0, 0)),
        ],
        out_shape=[
            jax.ShapeDtypeStruct((n, h), f32),
            jax.ShapeDtypeStruct((1, 1), f32),
        ],
        input_output_aliases={0: 0},
    )(pre)

    td = min(1024, n)
    kb = min(1024, h)
    recon = pl.pallas_call(
        _dec_body,
        grid=(n // td, h // kb),
        in_specs=[
            pl.BlockSpec((td, kb), lambda t, kk: (t, kk)),
            pl.BlockSpec((d, kb), lambda t, kk: (0, kk)),
            pl.BlockSpec((1, d), lambda t, kk: (0, 0)),
            pl.BlockSpec((1, d), lambda t, kk: (0, 0)),
        ],
        out_specs=pl.BlockSpec((td, d), lambda t, kk: (t, 0)),
        out_shape=jax.ShapeDtypeStruct((n, d), f32),
    )(hidden, wdec_bf16, bdec2, bpre2)

    tl = min(1024, n)
    loss_sum = pl.pallas_call(
        _loss_body,
        grid=(n // tl,),
        in_specs=[
            pl.BlockSpec((tl, d), lambda t: (t, 0)),
            pl.BlockSpec((tl, d), lambda t: (t, 0)),
        ],
        out_specs=pl.BlockSpec((1, 1), lambda t: (0, 0)),
        out_shape=jax.ShapeDtypeStruct((1, 1), f32),
    )(recon, x)

    rec_loss = loss_sum[0, 0] / jnp.float32(n * d)
    l0 = l0_sum[0, 0] / jnp.float32(n)
    sparsity = jnp.zeros((), f32)
    return (recon, hidden, rec_loss, rec_loss, sparsity, l0)
